# Initial kernel scaffold; baseline (speedup 1.0000x reference)
import jax
import jax.numpy as jnp
from jax.experimental import pallas as pl


def kernel(edge_index, nfeat, efeat, node_table, edge_tables, Ws, bs, gammas, betas, pW1, pb1, pW2, pb2):
    raise NotImplementedError("write your pallas kernel here")


# speedup vs baseline: 8.6698x; 8.6698x over previous
import functools

import jax
import jax.numpy as jnp
from jax import lax
from jax.experimental import pallas as pl
from jax.experimental.pallas import tpu as pltpu
from jax.experimental.pallas import tpu_sc as plsc

_N = 10000
_E = 320000
_D = 128
_NV = 100
_EV = 5
_NF = 9
_EF = 3
_NTAB = 1024
_NSC = 2
_NTILE = 16
_NW = _NSC * _NTILE
_K = 128
_NCH = 80
_EP = _NW * _NCH * _K
_NPAD = 10240
_RPT = _NPAD // _NTILE
_NB = 25
_BN = _N // _NB

_sc_mesh = plsc.VectorSubcoreMesh(core_axis_name="c", subcore_axis_name="s")


def _sc_scatter_body(h_hbm, src_hbm, dst_hbm, out_hbm,
                     src_v, dst_v, buf, zb, neigh, sem):
    c = lax.axis_index("c")
    s = lax.axis_index("s")
    w = c * _NTILE + s
    pltpu.sync_copy(src_hbm.at[w], src_v)
    pltpu.sync_copy(dst_hbm.at[w], dst_v)
    zeros = jnp.zeros((16,), jnp.float32)
    for r in range(16):
        for q in range(_D // 16):
            zb[r, pl.ds(q * 16, 16)] = zeros
    base = s * _RPT

    def zrow(k, carry):
        pltpu.sync_copy(zb, neigh.at[pl.ds(base + k * 16, 16)])
        return carry

    lax.fori_loop(0, _RPT // 16, zrow, 0)
    plsc.subcore_barrier()

    def chunk(j, carry):
        pltpu.async_copy(h_hbm.at[src_v.at[j]], buf, sem).wait()
        pltpu.sync_copy(buf, neigh.at[dst_v.at[j]], add=True)
        return carry

    lax.fori_loop(0, _NCH, chunk, 0)
    plsc.subcore_barrier()
    pltpu.sync_copy(neigh.at[pl.ds(base, _RPT)],
                    out_hbm.at[c, pl.ds(base, _RPT)])


_sc_scatter = pl.kernel(
    _sc_scatter_body,
    out_type=jax.ShapeDtypeStruct((_NSC, _NPAD, _D), jnp.float32),
    mesh=_sc_mesh,
    scratch_types=[
        pltpu.VMEM((_NCH, _K), jnp.int32),
        pltpu.VMEM((_NCH, _K), jnp.int32),
        pltpu.VMEM((_K, _D), jnp.float32),
        pltpu.VMEM((16, _D), jnp.float32),
        pltpu.VMEM_SHARED((_NPAD, _D), jnp.float32),
        pltpu.SemaphoreType.DMA,
    ],
    compiler_params=pltpu.CompilerParams(needs_layout_passes=False),
)


def _sc_hist_body(ohe_hbm, dst_hbm, out_hbm, dst_v, buf, zb, c_s, sem):
    c = lax.axis_index("c")
    s = lax.axis_index("s")
    w = c * _NTILE + s
    pltpu.sync_copy(dst_hbm.at[w], dst_v)
    zeros = jnp.zeros((16,), jnp.float32)
    for r in range(16):
        zb[r, :] = zeros
    base = s * _RPT

    def zrow(k, carry):
        pltpu.sync_copy(zb, c_s.at[pl.ds(base + k * 16, 16)])
        return carry

    lax.fori_loop(0, _RPT // 16, zrow, 0)
    plsc.subcore_barrier()

    def chunk(j, carry):
        pltpu.sync_copy(ohe_hbm.at[w, j], buf)
        pltpu.sync_copy(buf, c_s.at[dst_v.at[j]], add=True)
        return carry

    lax.fori_loop(0, _NCH, chunk, 0)
    plsc.subcore_barrier()
    pltpu.sync_copy(c_s.at[pl.ds(base, _RPT)],
                    out_hbm.at[c, pl.ds(base, _RPT)])


_sc_hist = pl.kernel(
    _sc_hist_body,
    out_type=jax.ShapeDtypeStruct((_NSC, _NPAD, 16), jnp.float32),
    mesh=_sc_mesh,
    scratch_types=[
        pltpu.VMEM((_NCH, _K), jnp.int32),
        pltpu.VMEM((_K, 16), jnp.float32),
        pltpu.VMEM((16, 16), jnp.float32),
        pltpu.VMEM_SHARED((_NPAD, 16), jnp.float32),
        pltpu.SemaphoreType.DMA,
    ],
    compiler_params=pltpu.CompilerParams(needs_layout_passes=False),
)


def _h0_body(nf_ref, tab_ref, out_ref):
    nf = nf_ref[...]
    iota = lax.broadcasted_iota(jnp.int32, (_BN, _NTAB), 1)
    acc = jnp.zeros((_BN, _NTAB), jnp.float32)
    for f in range(_NF):
        acc += (iota == nf[:, f][:, None]).astype(jnp.float32)
    out_ref[...] = jnp.dot(acc, tab_ref[...],
                           preferred_element_type=jnp.float32)


_h0_call = pl.pallas_call(
    _h0_body,
    grid=(_NB,),
    in_specs=[pl.BlockSpec((_BN, _NF), lambda i: (i, 0)),
              pl.BlockSpec((_NTAB, _D), lambda i: (0, 0))],
    out_specs=pl.BlockSpec((_BN, _D), lambda i: (i, 0)),
    out_shape=jax.ShapeDtypeStruct((_N, _D), jnp.float32),
)


def _dense_body(is_last, h_ref, parts_ref, cparts_ref, etbl_ref, w_ref,
                b_ref, g_ref, be_ref, *rest):
    if is_last:
        pw1_ref, pb1_ref, pw2_ref, pb2_ref, out_ref, h2_s, sum_s, sq_s, gs_s = rest
    else:
        out_ref, h2_s, sum_s, sq_s, gs_s = rest
    t = pl.program_id(0)
    i = pl.program_id(1)

    @pl.when(t == 0)
    def _pass_a():
        csum = cparts_ref[0] + cparts_ref[1]
        neigh = parts_ref[0] + parts_ref[1]
        degs = 1.0 + jnp.sum(csum, axis=1, keepdims=True) * (1.0 / 3.0)
        ce = jnp.dot(csum, etbl_ref[...], preferred_element_type=jnp.float32)
        x = (h_ref[...] + neigh + ce) / degs
        h2 = jnp.dot(x, w_ref[...], preferred_element_type=jnp.float32) + b_ref[...]
        h2_s[pl.ds(i * _BN, _BN), :] = h2
        cs = jnp.sum(h2, axis=0, keepdims=True)
        cq = jnp.sum(h2 * h2, axis=0, keepdims=True)

        @pl.when(i == 0)
        def _():
            sum_s[...] = cs
            sq_s[...] = cq

        @pl.when(i > 0)
        def _():
            sum_s[...] += cs
            sq_s[...] += cq

    @pl.when(t == 1)
    def _pass_b():
        h2 = h2_s[pl.ds(i * _BN, _BN), :]
        mu = sum_s[...] * (1.0 / _N)
        var = sq_s[...] * (1.0 / _N) - mu * mu
        y = (h2 - mu) / jnp.sqrt(var + 1e-5) * g_ref[...] + be_ref[...]
        hn = jnp.maximum(y, 0.0)
        if not is_last:
            out_ref[...] = hn
        else:
            cs = jnp.sum(hn, axis=0, keepdims=True)

            @pl.when(i == 0)
            def _():
                gs_s[...] = cs

            @pl.when(i > 0)
            def _():
                gs_s[...] += cs

            @pl.when(i == _NB - 1)
            def _():
                gvec = gs_s[...] * (1.0 / _N)
                z1 = jnp.maximum(
                    jnp.dot(gvec, pw1_ref[...],
                            preferred_element_type=jnp.float32) + pb1_ref[...],
                    0.0)
                out_ref[...] = jnp.dot(
                    z1, pw2_ref[...],
                    preferred_element_type=jnp.float32) + pb2_ref[...]


def _make_dense(is_last):
    in_specs = [
        pl.BlockSpec((_BN, _D), lambda t, i: (i, 0)),
        pl.BlockSpec((_NSC, _BN, _D), lambda t, i: (0, i, 0)),
        pl.BlockSpec((_NSC, _BN, 16), lambda t, i: (0, i, 0)),
        pl.BlockSpec((16, _D), lambda t, i: (0, 0)),
        pl.BlockSpec((_D, _D), lambda t, i: (0, 0)),
        pl.BlockSpec((1, _D), lambda t, i: (0, 0)),
        pl.BlockSpec((1, _D), lambda t, i: (0, 0)),
        pl.BlockSpec((1, _D), lambda t, i: (0, 0)),
    ]
    if is_last:
        in_specs += [
            pl.BlockSpec((_D, _D), lambda t, i: (0, 0)),
            pl.BlockSpec((1, _D), lambda t, i: (0, 0)),
            pl.BlockSpec((_D, _D), lambda t, i: (0, 0)),
            pl.BlockSpec((1, _D), lambda t, i: (0, 0)),
        ]
        out_specs = pl.BlockSpec((1, _D), lambda t, i: (0, 0))
        out_shape = jax.ShapeDtypeStruct((1, _D), jnp.float32)
    else:
        out_specs = pl.BlockSpec((_BN, _D), lambda t, i: (i, 0))
        out_shape = jax.ShapeDtypeStruct((_N, _D), jnp.float32)
    return pl.pallas_call(
        functools.partial(_dense_body, is_last),
        grid=(2, _NB),
        in_specs=in_specs,
        out_specs=out_specs,
        out_shape=out_shape,
        scratch_shapes=[
            pltpu.VMEM((_N, _D), jnp.float32),
            pltpu.VMEM((1, _D), jnp.float32),
            pltpu.VMEM((1, _D), jnp.float32),
            pltpu.VMEM((1, _D), jnp.float32),
        ],
    )


_dense_mid = _make_dense(False)
_dense_last = _make_dense(True)


def kernel(edge_index, nfeat, efeat, node_table, edge_tables, Ws, bs,
           gammas, betas, pW1, pb1, pW2, pb2):
    i32 = jnp.int32
    src = edge_index[0].astype(i32)
    dst = edge_index[1].astype(i32)
    srcp = jnp.pad(src, (0, _EP - _E)).reshape(_NW, _NCH, _K)
    dstp = jnp.pad(dst, (0, _EP - _E),
                   constant_values=_N).reshape(_NW, _NCH, _K)
    jf = (efeat.astype(i32)
          + (jnp.arange(_EF, dtype=i32) * _EV)[None, :])
    ohe = jnp.sum(jf[:, :, None] == jnp.arange(16, dtype=i32),
                  axis=1, dtype=jnp.float32)
    ohep = jnp.pad(ohe, ((0, _EP - _E), (0, 0))).reshape(_NW, _NCH, _K, 16)
    nfflat = nfeat.astype(i32) + (jnp.arange(_NF, dtype=i32) * _NV)[None, :]
    tabp = jnp.pad(node_table, ((0, _NTAB - _NF * _NV), (0, 0)))
    etblp = jnp.pad(edge_tables, ((0, 0), (0, 1), (0, 0)))
    b2 = bs.reshape(-1, 1, _D)
    gm = gammas.reshape(-1, 1, _D)
    bt = betas.reshape(-1, 1, _D)
    pb1r = pb1.reshape(1, _D)
    pw2p = jnp.pad(pW2, ((0, 0), (0, _D - pW2.shape[1])))
    pb2p = jnp.pad(pb2, (0, _D - pb2.shape[0])).reshape(1, _D)

    h = _h0_call(nfflat, tabp)
    cparts = _sc_hist(ohep, dstp)
    pre = None
    for l in range(3):
        parts = _sc_scatter(h, srcp, dstp)
        if l < 2:
            h = _dense_mid(h, parts, cparts, etblp[l], Ws[l],
                           b2[l], gm[l], bt[l])
        else:
            pre = _dense_last(h, parts, cparts, etblp[l], Ws[l],
                              b2[l], gm[l], bt[l], pW1, pb1r, pw2p, pb2p)
    return pre[:, :1]
